# baseline (device time: 123909 ns/iter reference)
import jax
import jax.numpy as jnp
from jax import lax
from jax.experimental import pallas as pl
from jax.experimental.pallas import tpu as pltpu

N_DEV = 16
BLK = 64
DH = 64


def kernel(x, Wq, K_ext, V_ext, Wo):
    B, Sq, D = x.shape
    Skv = K_ext.shape[1]
    HD = Wq.shape[1]
    H_per = HD // DH
    M = B * Sq

    pos = lax.axis_index("i")
    k_loc = lax.dynamic_slice_in_dim(K_ext, pos * H_per, H_per, axis=2)
    v_loc = lax.dynamic_slice_in_dim(V_ext, pos * H_per, H_per, axis=2)
    k_loc = jnp.moveaxis(k_loc, 2, 1)
    v_loc = jnp.moveaxis(v_loc, 2, 1)

    def body(x_ref, wq_ref, k_ref, v_ref, wo_ref, out_ref,
             comm_ref, ctx_ref, send_sems, recv_sems):
        my = lax.axis_index("i")
        left = lax.rem(my + N_DEV - 1, N_DEV)
        right = lax.rem(my + 1, N_DEV)

        barrier = pltpu.get_barrier_semaphore()
        pl.semaphore_signal(barrier, inc=1, device_id=(left,),
                            device_id_type=pl.DeviceIdType.MESH)
        pl.semaphore_signal(barrier, inc=1, device_id=(right,),
                            device_id_type=pl.DeviceIdType.MESH)
        pl.semaphore_wait(barrier, 2)

        x2 = x_ref[...].reshape(M, D)
        q = jnp.dot(x2, wq_ref[...], preferred_element_type=jnp.float32)

        rb = lax.broadcasted_iota(jnp.int32, (Sq, Skv), 0) // BLK
        cb = lax.broadcasted_iota(jnp.int32, (Sq, Skv), 1) // BLK
        mask = cb <= rb

        for b in range(B):
            for h in range(H_per):
                qh = q[b * Sq:(b + 1) * Sq, h * DH:(h + 1) * DH]
                kh = k_ref[b, h]
                vh = v_ref[b, h]
                s = lax.dot_general(
                    qh, kh, (((1,), (1,)), ((), ())),
                    preferred_element_type=jnp.float32,
                ) * 0.125
                s = jnp.where(mask, s, -1e9)
                m = jnp.max(s, axis=1, keepdims=True)
                w = jnp.exp(s - m)
                w = w / jnp.sum(w, axis=1, keepdims=True)
                ctx_ref[b * Sq:(b + 1) * Sq, h * DH:(h + 1) * DH] = jnp.dot(
                    w, vh, preferred_element_type=jnp.float32)

        partial = jnp.dot(ctx_ref[...], wo_ref[...],
                          preferred_element_type=jnp.float32)
        comm_ref[0] = partial
        acc = partial

        for h in range(N_DEV - 1):
            rdma = pltpu.make_async_remote_copy(
                src_ref=comm_ref.at[h],
                dst_ref=comm_ref.at[h + 1],
                send_sem=send_sems.at[h],
                recv_sem=recv_sems.at[h],
                device_id=(right,),
                device_id_type=pl.DeviceIdType.MESH,
            )
            rdma.start()
            rdma.wait()
            acc = acc + comm_ref[h + 1]

        out_ref[...] = acc.reshape(B, Sq, D)

    return pl.pallas_call(
        body,
        out_shape=jax.ShapeDtypeStruct((B, Sq, D), jnp.float32),
        in_specs=[pl.BlockSpec(memory_space=pltpu.VMEM)] * 5,
        out_specs=pl.BlockSpec(memory_space=pltpu.VMEM),
        scratch_shapes=[
            pltpu.VMEM((N_DEV, M, D), jnp.float32),
            pltpu.VMEM((M, HD), jnp.float32),
            pltpu.SemaphoreType.DMA((N_DEV - 1,)),
            pltpu.SemaphoreType.DMA((N_DEV - 1,)),
        ],
        compiler_params=pltpu.CompilerParams(collective_id=0),
    )(x, Wq, k_loc, v_loc, Wo)


# device time: 42584 ns/iter; 2.9098x vs baseline; 2.9098x over previous
import jax
import jax.numpy as jnp
from jax import lax
from jax.experimental import pallas as pl
from jax.experimental.pallas import tpu as pltpu

N_DEV = 16
BLK = 64
DH = 64


def kernel(x, Wq, K_ext, V_ext, Wo):
    B, Sq, D = x.shape
    Skv = K_ext.shape[1]
    HD = Wq.shape[1]
    H_per = HD // DH
    M = B * Sq

    pos = lax.axis_index("i")
    k_loc = lax.dynamic_slice_in_dim(K_ext, pos * H_per, H_per, axis=2)
    v_loc = lax.dynamic_slice_in_dim(V_ext, pos * H_per, H_per, axis=2)
    k_loc = jnp.moveaxis(k_loc, 2, 1)
    v_loc = jnp.moveaxis(v_loc, 2, 1)

    CHUNK = M // N_DEV

    def body(x_ref, wq_ref, k_ref, v_ref, wo_ref, out_ref,
             acc_ref, ctx_ref, rs0, rs1, rs2, rs3,
             send_sems, recv_sems):
        my = lax.axis_index("i")
        partners = [my ^ (1 << b) for b in range(4)]

        barrier = pltpu.get_barrier_semaphore()
        for p in partners:
            pl.semaphore_signal(barrier, inc=1, device_id=(p,),
                                device_id_type=pl.DeviceIdType.MESH)
        pl.semaphore_wait(barrier, 4)

        x2 = x_ref[...].reshape(M, D)
        q = jnp.dot(x2, wq_ref[...], preferred_element_type=jnp.float32)

        rb = lax.broadcasted_iota(jnp.int32, (Sq, Skv), 0) // BLK
        cb = lax.broadcasted_iota(jnp.int32, (Sq, Skv), 1) // BLK
        mask = cb <= rb

        for b in range(B):
            for h in range(H_per):
                qh = q[b * Sq:(b + 1) * Sq, h * DH:(h + 1) * DH]
                kh = k_ref[b, h]
                vh = v_ref[b, h]
                s = lax.dot_general(
                    qh, kh, (((1,), (1,)), ((), ())),
                    preferred_element_type=jnp.float32,
                ) * 0.125
                s = jnp.where(mask, s, -1e9)
                m = jnp.max(s, axis=1, keepdims=True)
                w = jnp.exp(s - m)
                w = w / jnp.sum(w, axis=1, keepdims=True)
                ctx_ref[b * Sq:(b + 1) * Sq, h * DH:(h + 1) * DH] = jnp.dot(
                    w, vh, preferred_element_type=jnp.float32)

        acc_ref[...] = jnp.dot(ctx_ref[...], wo_ref[...],
                               preferred_element_type=jnp.float32)

        rs_bufs = [rs0, rs1, rs2, rs3]
        lo = my * 0
        for k in range(4):
            b = 3 - k
            half = (M >> (k + 1))
            partner = my ^ (1 << b)
            bit = (my >> b) & 1
            new_lo = lo + bit * half
            send_lo = lo + (1 - bit) * half
            rdma = pltpu.make_async_remote_copy(
                src_ref=acc_ref.at[pl.ds(send_lo, half)],
                dst_ref=rs_bufs[k],
                send_sem=send_sems.at[k],
                recv_sem=recv_sems.at[k],
                device_id=(partner,),
                device_id_type=pl.DeviceIdType.MESH,
            )
            rdma.start()
            rdma.wait()
            acc_ref[pl.ds(new_lo, half), :] = (
                acc_ref[pl.ds(new_lo, half), :] + rs_bufs[k][...])
            lo = new_lo

        for b in range(4):
            n = CHUNK << b
            blk_lo = ((my >> b) << b) * CHUNK
            partner = my ^ (1 << b)
            rdma = pltpu.make_async_remote_copy(
                src_ref=acc_ref.at[pl.ds(blk_lo, n)],
                dst_ref=acc_ref.at[pl.ds(blk_lo, n)],
                send_sem=send_sems.at[4 + b],
                recv_sem=recv_sems.at[4 + b],
                device_id=(partner,),
                device_id_type=pl.DeviceIdType.MESH,
            )
            rdma.start()
            rdma.wait()

        out_ref[...] = acc_ref[...].reshape(B, Sq, D)

    return pl.pallas_call(
        body,
        out_shape=jax.ShapeDtypeStruct((B, Sq, D), jnp.float32),
        in_specs=[pl.BlockSpec(memory_space=pltpu.VMEM)] * 5,
        out_specs=pl.BlockSpec(memory_space=pltpu.VMEM),
        scratch_shapes=[
            pltpu.VMEM((M, D), jnp.float32),
            pltpu.VMEM((M, HD), jnp.float32),
            pltpu.VMEM((M // 2, D), jnp.float32),
            pltpu.VMEM((M // 4, D), jnp.float32),
            pltpu.VMEM((M // 8, D), jnp.float32),
            pltpu.VMEM((M // 16, D), jnp.float32),
            pltpu.SemaphoreType.DMA((8,)),
            pltpu.SemaphoreType.DMA((8,)),
        ],
        compiler_params=pltpu.CompilerParams(collective_id=0),
    )(x, Wq, k_loc, v_loc, Wo)


# device time: 27489 ns/iter; 4.5076x vs baseline; 1.5491x over previous
import jax
import jax.numpy as jnp
from jax import lax
from jax.experimental import pallas as pl
from jax.experimental.pallas import tpu as pltpu

N_DEV = 16
BLK = 64
DH = 64


def kernel(x, Wq, K_ext, V_ext, Wo):
    B, Sq, D = x.shape
    Skv = K_ext.shape[1]
    HD = Wq.shape[1]
    H_per = HD // DH
    M = B * Sq
    CHUNK = M // N_DEV

    pos = lax.axis_index("i")
    k_loc = lax.dynamic_slice_in_dim(K_ext, pos * H_per, H_per, axis=2)
    v_loc = lax.dynamic_slice_in_dim(V_ext, pos * H_per, H_per, axis=2)
    k_loc = jnp.moveaxis(k_loc, 2, 1)
    v_loc = jnp.moveaxis(v_loc, 2, 1)

    def body(x_ref, wq_ref, k_ref, v_ref, wo_ref, out_ref,
             acc_ref, ctx_ref, stage_ref,
             rs_send, rs_recv, ag_send, ag_recv):
        my = lax.axis_index("i")

        barrier = pltpu.get_barrier_semaphore()
        for j in range(1, N_DEV):
            p = lax.rem(my + j, N_DEV)
            pl.semaphore_signal(barrier, inc=1, device_id=(p,),
                                device_id_type=pl.DeviceIdType.MESH)
        pl.semaphore_wait(barrier, N_DEV - 1)

        x2 = x_ref[...].reshape(M, D)
        q = jnp.dot(x2, wq_ref[...], preferred_element_type=jnp.float32)

        rb = lax.broadcasted_iota(jnp.int32, (Sq, Skv), 0) // BLK
        cb = lax.broadcasted_iota(jnp.int32, (Sq, Skv), 1) // BLK
        mask = cb <= rb

        for b in range(B):
            for h in range(H_per):
                qh = q[b * Sq:(b + 1) * Sq, h * DH:(h + 1) * DH]
                kh = k_ref[b, h]
                vh = v_ref[b, h]
                s = lax.dot_general(
                    qh, kh, (((1,), (1,)), ((), ())),
                    preferred_element_type=jnp.float32,
                ) * 0.125
                s = jnp.where(mask, s, -1e9)
                m = jnp.max(s, axis=1, keepdims=True)
                w = jnp.exp(s - m)
                w = w / jnp.sum(w, axis=1, keepdims=True)
                ctx_ref[b * Sq:(b + 1) * Sq, h * DH:(h + 1) * DH] = jnp.dot(
                    w, vh, preferred_element_type=jnp.float32)

        acc_ref[...] = jnp.dot(ctx_ref[...], wo_ref[...],
                               preferred_element_type=jnp.float32)

        pending_sends = []

        r_rdmas = []
        for j in range(1, N_DEV):
            p = lax.rem(my + j, N_DEV)
            rdma = pltpu.make_async_remote_copy(
                src_ref=acc_ref.at[pl.ds(p * CHUNK, CHUNK)],
                dst_ref=stage_ref.at[j - 1],
                send_sem=rs_send.at[j - 1],
                recv_sem=rs_recv.at[j - 1],
                device_id=(p,),
                device_id_type=pl.DeviceIdType.MESH,
            )
            rdma.start()
            r_rdmas.append(rdma)
        for rdma in r_rdmas:
            rdma.wait_recv()
        pending_sends += r_rdmas

        my_lo = my * CHUNK
        acc_ref[pl.ds(my_lo, CHUNK), :] = (
            acc_ref[pl.ds(my_lo, CHUNK), :]
            + jnp.sum(stage_ref[...], axis=0))

        b_rdmas = []
        for j in range(1, N_DEV):
            p = lax.rem(my + j, N_DEV)
            rdma = pltpu.make_async_remote_copy(
                src_ref=acc_ref.at[pl.ds(my_lo, CHUNK)],
                dst_ref=acc_ref.at[pl.ds(my_lo, CHUNK)],
                send_sem=ag_send.at[j - 1],
                recv_sem=ag_recv.at[j - 1],
                device_id=(p,),
                device_id_type=pl.DeviceIdType.MESH,
            )
            rdma.start()
            b_rdmas.append(rdma)
        for rdma in b_rdmas:
            rdma.wait_recv()
        pending_sends += b_rdmas

        for rdma in pending_sends:
            rdma.wait_send()

        out_ref[...] = acc_ref[...].reshape(B, Sq, D)

    return pl.pallas_call(
        body,
        out_shape=jax.ShapeDtypeStruct((B, Sq, D), jnp.float32),
        in_specs=[pl.BlockSpec(memory_space=pltpu.VMEM)] * 5,
        out_specs=pl.BlockSpec(memory_space=pltpu.VMEM),
        scratch_shapes=[
            pltpu.VMEM((M, D), jnp.float32),
            pltpu.VMEM((M, HD), jnp.float32),
            pltpu.VMEM((N_DEV - 1, CHUNK, D), jnp.float32),
            pltpu.SemaphoreType.DMA((N_DEV - 1,)),
            pltpu.SemaphoreType.DMA((N_DEV - 1,)),
            pltpu.SemaphoreType.DMA((N_DEV - 1,)),
            pltpu.SemaphoreType.DMA((N_DEV - 1,)),
        ],
        compiler_params=pltpu.CompilerParams(collective_id=0),
    )(x, Wq, k_loc, v_loc, Wo)


# device time: 24883 ns/iter; 4.9797x vs baseline; 1.1047x over previous
import jax
import jax.numpy as jnp
from jax import lax
from jax.experimental import pallas as pl
from jax.experimental.pallas import tpu as pltpu

N_DEV = 16
BLK = 64
DH = 64


def kernel(x, Wq, K_ext, V_ext, Wo):
    B, Sq, D = x.shape
    Skv = K_ext.shape[1]
    HD = Wq.shape[1]
    H_per = HD // DH
    M = B * Sq
    CHUNK = M // N_DEV

    pos = lax.axis_index("i")
    k_loc = lax.dynamic_slice_in_dim(K_ext, pos * H_per, H_per, axis=2)
    v_loc = lax.dynamic_slice_in_dim(V_ext, pos * H_per, H_per, axis=2)
    k_loc = jnp.moveaxis(k_loc, 2, 1)
    v_loc = jnp.moveaxis(v_loc, 2, 1)

    def body(x_ref, wq_ref, k_ref, v_ref, wo_ref, out_ref,
             acc_ref, ctx_ref, stage_ref,
             rs_send, rs_recv, ag_send, ag_recv):
        my = lax.axis_index("i")

        barrier = pltpu.get_barrier_semaphore()
        for j in range(1, N_DEV):
            p = lax.rem(my + j, N_DEV)
            pl.semaphore_signal(barrier, inc=1, device_id=(p,),
                                device_id_type=pl.DeviceIdType.MESH)

        x2 = x_ref[...].reshape(M, D)
        q = jnp.dot(x2, wq_ref[...], preferred_element_type=jnp.float32)

        rb = lax.broadcasted_iota(jnp.int32, (Sq, Skv), 0) // BLK
        cb = lax.broadcasted_iota(jnp.int32, (Sq, Skv), 1) // BLK
        mask = cb <= rb

        for b in range(B):
            for h in range(H_per):
                qh = q[b * Sq:(b + 1) * Sq, h * DH:(h + 1) * DH]
                kh = k_ref[b, h]
                vh = v_ref[b, h]
                s = lax.dot_general(
                    qh, kh, (((1,), (1,)), ((), ())),
                    preferred_element_type=jnp.float32,
                ) * 0.125
                s = jnp.where(mask, s, -1e9)
                m = jnp.max(s, axis=1, keepdims=True)
                w = jnp.exp(s - m)
                w = w / jnp.sum(w, axis=1, keepdims=True)
                ctx_ref[b * Sq:(b + 1) * Sq, h * DH:(h + 1) * DH] = jnp.dot(
                    w, vh, preferred_element_type=jnp.float32)

        acc_ref[...] = jnp.dot(ctx_ref[...], wo_ref[...],
                               preferred_element_type=jnp.float32)

        pl.semaphore_wait(barrier, N_DEV - 1)
        pending_sends = []

        r_rdmas = []
        for j in range(1, N_DEV):
            p = lax.rem(my + j, N_DEV)
            rdma = pltpu.make_async_remote_copy(
                src_ref=acc_ref.at[pl.ds(p * CHUNK, CHUNK)],
                dst_ref=stage_ref.at[j - 1],
                send_sem=rs_send.at[j - 1],
                recv_sem=rs_recv.at[j - 1],
                device_id=(p,),
                device_id_type=pl.DeviceIdType.MESH,
            )
            rdma.start()
            r_rdmas.append(rdma)
        for rdma in r_rdmas:
            rdma.wait_recv()
        pending_sends += r_rdmas

        my_lo = my * CHUNK
        acc_ref[pl.ds(my_lo, CHUNK), :] = (
            acc_ref[pl.ds(my_lo, CHUNK), :]
            + jnp.sum(stage_ref[...], axis=0))

        b_rdmas = []
        for j in range(1, N_DEV):
            p = lax.rem(my + j, N_DEV)
            rdma = pltpu.make_async_remote_copy(
                src_ref=acc_ref.at[pl.ds(my_lo, CHUNK)],
                dst_ref=acc_ref.at[pl.ds(my_lo, CHUNK)],
                send_sem=ag_send.at[j - 1],
                recv_sem=ag_recv.at[j - 1],
                device_id=(p,),
                device_id_type=pl.DeviceIdType.MESH,
            )
            rdma.start()
            b_rdmas.append(rdma)
        for rdma in b_rdmas:
            rdma.wait_recv()
        pending_sends += b_rdmas

        for rdma in pending_sends:
            rdma.wait_send()

        out_ref[...] = acc_ref[...].reshape(B, Sq, D)

    return pl.pallas_call(
        body,
        out_shape=jax.ShapeDtypeStruct((B, Sq, D), jnp.float32),
        in_specs=[pl.BlockSpec(memory_space=pltpu.VMEM)] * 5,
        out_specs=pl.BlockSpec(memory_space=pltpu.VMEM),
        scratch_shapes=[
            pltpu.VMEM((M, D), jnp.float32),
            pltpu.VMEM((M, HD), jnp.float32),
            pltpu.VMEM((N_DEV - 1, CHUNK, D), jnp.float32),
            pltpu.SemaphoreType.DMA((N_DEV - 1,)),
            pltpu.SemaphoreType.DMA((N_DEV - 1,)),
            pltpu.SemaphoreType.DMA((N_DEV - 1,)),
            pltpu.SemaphoreType.DMA((N_DEV - 1,)),
        ],
        compiler_params=pltpu.CompilerParams(collective_id=0),
    )(x, Wq, k_loc, v_loc, Wo)


# device time: 20571 ns/iter; 6.0235x vs baseline; 1.2096x over previous
import jax
import jax.numpy as jnp
from jax import lax
from jax.experimental import pallas as pl
from jax.experimental.pallas import tpu as pltpu

N_DEV = 16
BLK = 64
DH = 64


def kernel(x, Wq, K_ext, V_ext, Wo):
    B, Sq, D = x.shape
    Skv = K_ext.shape[1]
    HD = Wq.shape[1]
    H_per = HD // DH
    M = B * Sq
    CHUNK = M // N_DEV

    pos = lax.axis_index("i")
    k_loc = lax.dynamic_slice_in_dim(K_ext, pos * H_per, H_per, axis=2)
    v_loc = lax.dynamic_slice_in_dim(V_ext, pos * H_per, H_per, axis=2)
    k_loc = jnp.moveaxis(k_loc, 2, 1)
    v_loc = jnp.moveaxis(v_loc, 2, 1)

    def body(x_ref, wq_ref, k_ref, v_ref, wo_ref, out_ref,
             acc_ref, ctx_ref, comm_ref, stage_ref,
             rs_send, rs_recv, ag_send, ag_recv):
        my = lax.axis_index("i")

        barrier = pltpu.get_barrier_semaphore()
        for j in range(1, N_DEV):
            p = lax.rem(my + j, N_DEV)
            pl.semaphore_signal(barrier, inc=1, device_id=(p,),
                                device_id_type=pl.DeviceIdType.MESH)

        x2 = x_ref[...].reshape(M, D)
        q = jnp.dot(x2, wq_ref[...], preferred_element_type=jnp.float32)

        rb = lax.broadcasted_iota(jnp.int32, (Sq, Skv), 0) // BLK
        cb = lax.broadcasted_iota(jnp.int32, (Sq, Skv), 1) // BLK
        mask = cb <= rb

        for b in range(B):
            for h in range(H_per):
                qh = q[b * Sq:(b + 1) * Sq, h * DH:(h + 1) * DH]
                kh = k_ref[b, h]
                vh = v_ref[b, h]
                s = lax.dot_general(
                    qh, kh, (((1,), (1,)), ((), ())),
                    preferred_element_type=jnp.float32,
                ) * 0.125
                s = jnp.where(mask, s, -1e9)
                m = jnp.max(s, axis=1, keepdims=True)
                w = jnp.exp(s - m)
                w = w / jnp.sum(w, axis=1, keepdims=True)
                ctx_ref[b * Sq:(b + 1) * Sq, h * DH:(h + 1) * DH] = jnp.dot(
                    w, vh, preferred_element_type=jnp.float32)

        acc_ref[...] = jnp.dot(ctx_ref[...], wo_ref[...],
                               preferred_element_type=jnp.float32)
        comm_ref[...] = acc_ref[...].astype(jnp.bfloat16)

        pl.semaphore_wait(barrier, N_DEV - 1)
        pending_sends = []

        r_rdmas = []
        for j in range(1, N_DEV):
            p = lax.rem(my + j, N_DEV)
            rdma = pltpu.make_async_remote_copy(
                src_ref=comm_ref.at[pl.ds(p * CHUNK, CHUNK)],
                dst_ref=stage_ref.at[j - 1],
                send_sem=rs_send.at[j - 1],
                recv_sem=rs_recv.at[j - 1],
                device_id=(p,),
                device_id_type=pl.DeviceIdType.MESH,
            )
            rdma.start()
            r_rdmas.append(rdma)
        for rdma in r_rdmas:
            rdma.wait_recv()
        pending_sends += r_rdmas

        my_lo = my * CHUNK
        reduced = (acc_ref[pl.ds(my_lo, CHUNK), :]
                   + jnp.sum(stage_ref[...].astype(jnp.float32), axis=0))
        comm_ref[pl.ds(my_lo, CHUNK), :] = reduced.astype(jnp.bfloat16)

        b_rdmas = []
        for j in range(1, N_DEV):
            p = lax.rem(my + j, N_DEV)
            rdma = pltpu.make_async_remote_copy(
                src_ref=comm_ref.at[pl.ds(my_lo, CHUNK)],
                dst_ref=comm_ref.at[pl.ds(my_lo, CHUNK)],
                send_sem=ag_send.at[j - 1],
                recv_sem=ag_recv.at[j - 1],
                device_id=(p,),
                device_id_type=pl.DeviceIdType.MESH,
            )
            rdma.start()
            b_rdmas.append(rdma)
        for rdma in b_rdmas:
            rdma.wait_recv()
        pending_sends += b_rdmas

        for rdma in pending_sends:
            rdma.wait_send()

        out_ref[...] = comm_ref[...].astype(jnp.float32).reshape(B, Sq, D)

    return pl.pallas_call(
        body,
        out_shape=jax.ShapeDtypeStruct((B, Sq, D), jnp.float32),
        in_specs=[pl.BlockSpec(memory_space=pltpu.VMEM)] * 5,
        out_specs=pl.BlockSpec(memory_space=pltpu.VMEM),
        scratch_shapes=[
            pltpu.VMEM((M, D), jnp.float32),
            pltpu.VMEM((M, HD), jnp.float32),
            pltpu.VMEM((M, D), jnp.bfloat16),
            pltpu.VMEM((N_DEV - 1, CHUNK, D), jnp.bfloat16),
            pltpu.SemaphoreType.DMA((N_DEV - 1,)),
            pltpu.SemaphoreType.DMA((N_DEV - 1,)),
            pltpu.SemaphoreType.DMA((N_DEV - 1,)),
            pltpu.SemaphoreType.DMA((N_DEV - 1,)),
        ],
        compiler_params=pltpu.CompilerParams(collective_id=0),
    )(x, Wq, k_loc, v_loc, Wo)
